# baseline (speedup 1.0000x reference)
import functools
import math

import jax
import jax.numpy as jnp
from jax import lax
from jax.experimental import pallas as pl
from jax.experimental.pallas import tpu as pltpu
from jax.experimental.pallas import tpu_sc as plsc

NC = 2
NS = 16
L = 16
NW = NC * NS

B = 4096 * 200
D = 64
PER_W = B // NW
CHUNK = 128
NCHUNK = PER_W // CHUNK
SCALE = math.sqrt(D)

_mesh = plsc.VectorSubcoreMesh(core_axis_name="c", subcore_axis_name="s")


@functools.partial(
    pl.kernel,
    out_type=jax.ShapeDtypeStruct((B, D), jnp.float32),
    mesh=_mesh,
    scratch_types=[
        pltpu.VMEM((NCHUNK, CHUNK), jnp.int32),
        pltpu.VMEM((CHUNK, D), jnp.float32),
        pltpu.SemaphoreType.DMA,
    ],
    compiler_params=pltpu.CompilerParams(use_tc_tiling_on_sc=False),
)
def _embed(x_hbm, table_hbm, out_hbm, idx_v, rows_v, sem):
    wid = lax.axis_index("s") * NC + lax.axis_index("c")
    pltpu.sync_copy(x_hbm.at[wid], idx_v)

    def chunk_body(j, carry):
        pltpu.async_copy(table_hbm.at[idx_v.at[j]], rows_v, sem).wait()

        def scale_row(r, c2):
            for c in range(D // L):
                sl = pl.ds(c * L, L)
                rows_v[r, sl] = rows_v[r, sl] * SCALE
            return c2

        lax.fori_loop(0, CHUNK, scale_row, 0)
        pltpu.sync_copy(
            rows_v, out_hbm.at[pl.ds(wid * PER_W + j * CHUNK, CHUNK)]
        )
        return carry

    lax.fori_loop(0, NCHUNK, chunk_body, 0)


def kernel(x, table):
    x3 = x.reshape(NW, NCHUNK, CHUNK)
    out = _embed(x3, table)
    return out.reshape(4096, 200, D)


# speedup vs baseline: 1.2076x; 1.2076x over previous
import functools
import math

import jax
import jax.numpy as jnp
from jax import lax
from jax.experimental import pallas as pl
from jax.experimental.pallas import tpu as pltpu
from jax.experimental.pallas import tpu_sc as plsc

NC = 2
NS = 16
L = 16
NW = NC * NS

B = 4096 * 200
D = 64
PER_W = B // NW
CHUNK = 128
NCHUNK = PER_W // CHUNK
NG = 4
NGROUP = NCHUNK // NG
SCALE = math.sqrt(D)

_mesh = plsc.VectorSubcoreMesh(core_axis_name="c", subcore_axis_name="s")


@functools.partial(
    pl.kernel,
    out_type=jax.ShapeDtypeStruct((B, D), jnp.float32),
    mesh=_mesh,
    scratch_types=[
        pltpu.VMEM((NCHUNK, CHUNK), jnp.int32),
        pltpu.VMEM((NG, CHUNK, D), jnp.float32),
        pltpu.VMEM((NG, CHUNK, D), jnp.float32),
    ]
    + [pltpu.SemaphoreType.DMA] * (2 * NG),
    compiler_params=pltpu.CompilerParams(use_tc_tiling_on_sc=False),
)
def _embed(x_hbm, table_hbm, out_hbm, idx_v, g_v, o_v, *sems):
    gsem, osem = sems[:NG], sems[NG:]
    wid = lax.axis_index("s") * NC + lax.axis_index("c")
    base = wid * PER_W
    pltpu.sync_copy(x_hbm.at[wid], idx_v)

    for b in range(NG):
        pltpu.async_copy(table_hbm.at[idx_v.at[b]], g_v.at[b], gsem[b])

    def group(g, carry):
        for b in range(NG):
            j = g * NG + b
            pltpu.make_async_copy(
                table_hbm.at[idx_v.at[j]], g_v.at[b], gsem[b]
            ).wait()

            @pl.when(g > 0)
            def _():
                pltpu.make_async_copy(
                    o_v.at[b], out_hbm.at[pl.ds(base, CHUNK)], osem[b]
                ).wait()

            def srow(k, c2, b=b):
                r0 = k * 16
                for dr in range(16):
                    for c in range(D // L):
                        sl = pl.ds(c * L, L)
                        o_v[b, r0 + dr, sl] = g_v[b, r0 + dr, sl] * SCALE
                return c2

            lax.fori_loop(0, CHUNK // 16, srow, 0)

            pltpu.async_copy(
                o_v.at[b], out_hbm.at[pl.ds(base + j * CHUNK, CHUNK)], osem[b]
            )

            @pl.when(g < NGROUP - 1)
            def _():
                pltpu.async_copy(
                    table_hbm.at[idx_v.at[j + NG]], g_v.at[b], gsem[b]
                )
        return carry

    lax.fori_loop(0, NGROUP, group, 0)

    for b in range(NG):
        pltpu.make_async_copy(
            o_v.at[b], out_hbm.at[pl.ds(base, CHUNK)], osem[b]
        ).wait()


def kernel(x, table):
    x3 = x.reshape(NW, NCHUNK, CHUNK)
    out = _embed(x3, table)
    return out.reshape(4096, 200, D)
